# Initial kernel scaffold; baseline (speedup 1.0000x reference)
import jax
import jax.numpy as jnp
from jax.experimental import pallas as pl


def kernel(input):
    raise NotImplementedError("write your pallas kernel here")


# speedup vs baseline: 4.4924x; 4.4924x over previous
import jax
import jax.numpy as jnp
from jax import lax
from jax.experimental import pallas as pl
from jax.experimental.pallas import tpu as pltpu
from jax.experimental.pallas import tpu_sc as plsc

NUM_SPIKES = 128
LANES = 16
CHUNK = 112


def _spike_body(x_hbm, r_hbm, out_hbm, xv, rv, ov):
    B, C, P = x_hbm.shape
    n_workers = 32
    wpb = n_workers // B
    cols_pw = P // wpb
    n_chunks = cols_pw // CHUNK
    groups = CHUNK // LANES

    wid = lax.axis_index("s") * 2 + lax.axis_index("c")
    b = wid // wpb
    base = (wid % wpb) * cols_pw

    def chunk_body(j, _):
        pbase = base + j * CHUNK
        pltpu.sync_copy(x_hbm.at[b, :, pl.ds(pbase, CHUNK)], xv)
        pltpu.sync_copy(r_hbm.at[b, :, pl.ds(pbase, CHUNK)], rv)

        for g in range(groups):
            sl = pl.ds(g * LANES, LANES)
            lanecol = lax.iota(jnp.int32, LANES) + (g * LANES)

            def cum_body(c, acc):
                acc = acc + xv[c, sl]
                xv[c, sl] = acc
                return acc

            total = lax.fori_loop(0, C, cum_body, jnp.zeros((LANES,), jnp.float32))

            def spike_body(s, _):
                v = rv[s, sl] * total
                pos = jnp.zeros((LANES,), jnp.int32)
                for k in (64, 32, 16, 8, 4, 2, 1):
                    probe = plsc.load_gather(xv, [pos + (k - 1), lanecol])
                    pos = pos + jnp.where(probe < v, k, 0).astype(jnp.int32)
                ov[s, sl] = pos
                return 0

            lax.fori_loop(0, NUM_SPIKES, spike_body, 0)

        pltpu.sync_copy(ov, out_hbm.at[b, :, pl.ds(pbase, CHUNK)])
        return 0

    lax.fori_loop(0, n_chunks, chunk_body, 0)


def kernel(input):
    B, C, H, W = input.shape
    P = H * W
    x = input.reshape(B, C, P)
    rkey = jax.random.key(42)
    rand = jax.random.uniform(rkey, (B, NUM_SPIKES, H, W), dtype=input.dtype)
    rand = rand.reshape(B, NUM_SPIKES, P)

    mesh = plsc.VectorSubcoreMesh(
        core_axis_name="c", subcore_axis_name="s", num_cores=2, num_subcores=16
    )
    run = pl.kernel(
        _spike_body,
        out_type=jax.ShapeDtypeStruct((B, NUM_SPIKES, P), jnp.int32),
        mesh=mesh,
        scratch_types=[
            pltpu.VMEM((C, CHUNK), jnp.float32),
            pltpu.VMEM((NUM_SPIKES, CHUNK), jnp.float32),
            pltpu.VMEM((NUM_SPIKES, CHUNK), jnp.int32),
        ],
        compiler_params=pltpu.CompilerParams(use_tc_tiling_on_sc=False,
                                             needs_layout_passes=False),
    )
    out = run(x, rand)
    return out.reshape(B, NUM_SPIKES, H, W)


# speedup vs baseline: 13.1000x; 2.9160x over previous
import jax
import jax.numpy as jnp
from jax import lax
from jax.experimental import pallas as pl
from jax.experimental.pallas import tpu as pltpu
from jax.experimental.pallas import tpu_sc as plsc

NUM_SPIKES = 128
LANES = 16
CHUNK = 112
NBUF = 2


def _spike_body(x_hbm, r_hbm, out_hbm, xvs, rvs, ovs, sin, sout):
    B, C, P = x_hbm.shape
    n_workers = 32
    wpb = n_workers // B
    cols_pw = P // wpb
    n_chunks = cols_pw // CHUNK
    groups = CHUNK // LANES

    wid = lax.axis_index("s") * 2 + lax.axis_index("c")
    b = wid // wpb
    base = (wid % wpb) * cols_pw

    def in_slices(j):
        sl = pl.ds(base + j * CHUNK, CHUNK)
        return x_hbm.at[b, :, sl], r_hbm.at[b, :, sl]

    def issue_in(j, t):
        xs, rs = in_slices(j)
        pltpu.async_copy(xs, xvs[t], sin[t])
        pltpu.async_copy(rs, rvs[t], sin[t])

    def wait_in(j, t):
        xs, rs = in_slices(j)
        pltpu.make_async_copy(xs, xvs[t], sin[t]).wait()
        pltpu.make_async_copy(rs, rvs[t], sin[t]).wait()

    def out_slice(j):
        return out_hbm.at[b, :, pl.ds(base + j * CHUNK, CHUNK)]

    issue_in(0, 0)

    def chunk_pair(j2, _):
        for t in range(NBUF):
            j = j2 * NBUF + t
            xv, rv, ov = xvs[t], rvs[t], ovs[t]

            @pl.when(j < n_chunks)
            def _():
                @pl.when(j + 1 < n_chunks)
                def _():
                    issue_in(j + 1, (t + 1) % NBUF)

                wait_in(j, t)

                @pl.when(j >= NBUF)
                def _():
                    pltpu.make_async_copy(ov, out_slice(j - NBUF), sout[t]).wait()

                for g in range(groups):
                    sl = pl.ds(g * LANES, LANES)
                    lanecol = lax.iota(jnp.int32, LANES) + (g * LANES)

                    def cum_body(c, acc):
                        acc = acc + xv[c, sl]
                        xv[c, sl] = acc
                        return acc

                    total = plsc.parallel_loop(
                        0, C, carry=jnp.zeros((LANES,), jnp.float32), unroll=8
                    )(cum_body)

                    @plsc.parallel_loop(0, NUM_SPIKES, unroll=4)
                    def spike_body(s):
                        v = rv[s, sl] * total
                        pos = jnp.zeros((LANES,), jnp.int32)
                        for k in (64, 32, 16, 8, 4, 2, 1):
                            probe = plsc.load_gather(xv, [pos + (k - 1), lanecol])
                            pos = pos + jnp.where(probe < v, k, 0).astype(jnp.int32)
                        ov[s, sl] = pos

                pltpu.async_copy(ov, out_slice(j), sout[t])
        return 0

    lax.fori_loop(0, (n_chunks + NBUF - 1) // NBUF, chunk_pair, 0)

    for t in range(NBUF):
        j = n_chunks - NBUF + t
        buf = j % NBUF
        pltpu.make_async_copy(ovs[buf], out_slice(j), sout[buf]).wait()


def kernel(input):
    B, C, H, W = input.shape
    P = H * W
    x = input.reshape(B, C, P)
    rkey = jax.random.key(42)
    rand = jax.random.uniform(rkey, (B, NUM_SPIKES, P), dtype=input.dtype)

    mesh = plsc.VectorSubcoreMesh(
        core_axis_name="c", subcore_axis_name="s", num_cores=2, num_subcores=16
    )
    run = pl.kernel(
        _spike_body,
        out_type=jax.ShapeDtypeStruct((B, NUM_SPIKES, P), jnp.int32),
        mesh=mesh,
        scratch_types=[
            [pltpu.VMEM((C, CHUNK), jnp.float32) for _ in range(NBUF)],
            [pltpu.VMEM((NUM_SPIKES, CHUNK), jnp.float32) for _ in range(NBUF)],
            [pltpu.VMEM((NUM_SPIKES, CHUNK), jnp.int32) for _ in range(NBUF)],
            [pltpu.SemaphoreType.DMA for _ in range(NBUF)],
            [pltpu.SemaphoreType.DMA for _ in range(NBUF)],
        ],
        compiler_params=pltpu.CompilerParams(use_tc_tiling_on_sc=False,
                                             needs_layout_passes=False),
    )
    out = run(x, rand)
    return out.reshape(B, NUM_SPIKES, H, W)
